# baseline (speedup 1.0000x reference)
import functools

import jax
import jax.numpy as jnp
from jax import lax
from jax.experimental import pallas as pl
from jax.experimental.pallas import tpu as pltpu
from jax.experimental.pallas import tpu_sc as plsc

N_LAYERS = 3
NC, NS = 2, 16
NW = NC * NS
CHUNK = 128
HALF = 40
PAD_DST = 10000


def _mm_body(h_ref, w_ref, o_ref):
    o_ref[...] = jnp.dot(h_ref[...], w_ref[...],
                         preferred_element_type=jnp.float32)


def _matmul(h, w, br):
    n, d = h.shape
    return pl.pallas_call(
        _mm_body,
        grid=(n // br,),
        in_specs=[pl.BlockSpec((br, d), lambda i: (i, 0)),
                  pl.BlockSpec((d, d), lambda i: (0, 0))],
        out_specs=pl.BlockSpec((br, d), lambda i: (i, 0)),
        out_shape=jax.ShapeDtypeStruct((n, d), jnp.float32),
    )(h, w)


def _gru_body(p_ref, h_ref, wih_ref, whh_ref, bih_ref, bhh_ref, o_ref):
    d = h_ref.shape[1]
    agg = p_ref[0] + p_ref[1]
    h = h_ref[...]
    gi = jnp.dot(agg, wih_ref[...], preferred_element_type=jnp.float32)
    gi = gi + bih_ref[...]
    gh = jnp.dot(h, whh_ref[...], preferred_element_type=jnp.float32)
    gh = gh + bhh_ref[...]
    r = jax.nn.sigmoid(gi[:, :d] + gh[:, :d])
    z = jax.nn.sigmoid(gi[:, d:2 * d] + gh[:, d:2 * d])
    n = jnp.tanh(gi[:, 2 * d:] + r * gh[:, 2 * d:])
    o_ref[...] = (1.0 - z) * n + z * h


def _gru(partial, h, wih_t, whh_t, bih, bhh, br):
    n, d = h.shape
    return pl.pallas_call(
        _gru_body,
        grid=(n // br,),
        in_specs=[pl.BlockSpec((2, br, d), lambda i: (0, i, 0)),
                  pl.BlockSpec((br, d), lambda i: (i, 0)),
                  pl.BlockSpec((d, 3 * d), lambda i: (0, 0)),
                  pl.BlockSpec((d, 3 * d), lambda i: (0, 0)),
                  pl.BlockSpec((1, 3 * d), lambda i: (0, 0)),
                  pl.BlockSpec((1, 3 * d), lambda i: (0, 0))],
        out_specs=pl.BlockSpec((br, d), lambda i: (i, 0)),
        out_shape=jax.ShapeDtypeStruct((n, d), jnp.float32),
    )(partial, h, wih_t, whh_t, bih, bhh)


ZROWS = 16


def _seg_sum_body(n_nodes, d, base, rem,
                  m_hbm, src_hbm, dst_hbm, out_hbm,
                  src_v, dst_v, rows_a, rows_b, zbuf, acc, sem_a, sem_b):
    c = lax.axis_index("c")
    s = lax.axis_index("s")
    wid = c * NS + s

    zeros16 = jnp.zeros((16,), jnp.float32)

    def zrow(i, carry):
        def zcol(j, carry2):
            zbuf[i, pl.ds(j * 16, 16)] = zeros16
            return carry2
        return lax.fori_loop(0, d // 16, zcol, carry)

    lax.fori_loop(0, ZROWS, zrow, 0)
    for k in range(base // ZROWS):
        pltpu.sync_copy(zbuf, acc.at[pl.ds(s * base + k * ZROWS, ZROWS)])

    @pl.when(s == NS - 1)
    def _():
        pltpu.sync_copy(zbuf.at[pl.ds(0, rem)],
                        acc.at[pl.ds(NS * base, rem)])

    plsc.subcore_barrier()

    for t in range(2):
        pltpu.sync_copy(src_hbm.at[wid, t], src_v)
        pltpu.sync_copy(dst_hbm.at[wid, t], dst_v)
        def chunk_body(j, carry):
            pltpu.async_copy(m_hbm.at[src_v.at[j]], rows_a, sem_a).wait()
            pltpu.sync_copy(rows_a, acc.at[dst_v.at[j]], add=True)
            return carry

        lax.fori_loop(0, HALF, chunk_body, 0)
    plsc.subcore_barrier()

    pltpu.sync_copy(acc.at[pl.ds(s * base, base)],
                    out_hbm.at[c, pl.ds(s * base, base)])

    @pl.when(s == NS - 1)
    def _():
        pltpu.sync_copy(acc.at[pl.ds(NS * base, rem)],
                        out_hbm.at[c, pl.ds(NS * base, rem)])


def _seg_sum(m, src_r, dst_r):
    n_nodes, d = m.shape
    base = (n_nodes // (NS * 8)) * 8
    rem = n_nodes - NS * base
    assert base % ZROWS == 0 and rem <= ZROWS
    acc_rows = n_nodes + 8
    mesh = plsc.VectorSubcoreMesh(core_axis_name="c", subcore_axis_name="s")
    f = pl.kernel(
        functools.partial(_seg_sum_body, n_nodes, d, base, rem),
        out_type=jax.ShapeDtypeStruct((NC, n_nodes, d), jnp.float32),
        mesh=mesh,
        scratch_types=[
            pltpu.VMEM((HALF, CHUNK), jnp.int32),
            pltpu.VMEM((HALF, CHUNK), jnp.int32),
            pltpu.VMEM((CHUNK, d), jnp.float32),
            pltpu.VMEM((CHUNK, d), jnp.float32),
            pltpu.VMEM((ZROWS, d), jnp.float32),
            pltpu.VMEM_SHARED((acc_rows, d), jnp.float32),
            pltpu.SemaphoreType.DMA,
            pltpu.SemaphoreType.DMA,
        ],
    )
    return f(m, src_r, dst_r)


def _gather_body(h_hbm, map_hbm, out_hbm, idx_v, rows_v, sem):
    c = lax.axis_index("c")
    s = lax.axis_index("s")
    wid = c * NS + s
    pltpu.sync_copy(map_hbm.at[wid], idx_v)
    pltpu.async_copy(h_hbm.at[idx_v], rows_v, sem).wait()
    pltpu.sync_copy(rows_v, out_hbm.at[wid])


def _gather_clicked(h, mapping_idx):
    batch, num_clicked = mapping_idx.shape
    d = h.shape[1]
    mesh = plsc.VectorSubcoreMesh(core_axis_name="c", subcore_axis_name="s")
    f = pl.kernel(
        _gather_body,
        out_type=jax.ShapeDtypeStruct((batch, num_clicked, d), jnp.float32),
        mesh=mesh,
        scratch_types=[
            pltpu.VMEM((num_clicked,), jnp.int32),
            pltpu.VMEM((num_clicked, d), jnp.float32),
            pltpu.SemaphoreType.DMA,
        ],
    )
    return f(h, mapping_idx)


@jax.jit
def kernel(x, weight, w_ih, w_hh, b_ih, b_hh, edge_index, mapping_idx):
    n_nodes, d = x.shape
    n_edges = edge_index.shape[1]
    ept = 2 * HALF * CHUNK
    n_pad = NW * ept - n_edges

    src_r = jnp.concatenate(
        [edge_index[0], jnp.zeros((n_pad,), jnp.int32)]
    ).reshape(NW, 2, HALF, CHUNK)
    dst_r = jnp.concatenate(
        [edge_index[1], jnp.full((n_pad,), PAD_DST, jnp.int32)]
    ).reshape(NW, 2, HALF, CHUNK)
    wih_t = w_ih.T
    whh_t = w_hh.T
    bih = b_ih.reshape(1, 3 * d)
    bhh = b_hh.reshape(1, 3 * d)

    br = 2000
    h = x
    for i in range(N_LAYERS):
        m = _matmul(h, weight[i], br)
        partial = _seg_sum(m, src_r, dst_r)
        h = _gru(partial, h, wih_t, whh_t, bih, bhh, br)
    return _gather_clicked(h, mapping_idx)


# speedup vs baseline: 2.6633x; 2.6633x over previous
import functools

import jax
import jax.numpy as jnp
from jax import lax
from jax.experimental import pallas as pl
from jax.experimental.pallas import tpu as pltpu
from jax.experimental.pallas import tpu_sc as plsc

N_LAYERS = 3
NC, NS = 2, 16
NW = NC * NS
CHUNK = 128
HALF = 40
PAD_DST = 10000


def _mm_body(h_ref, w_ref, o_ref):
    o_ref[...] = jnp.dot(h_ref[...], w_ref[...],
                         preferred_element_type=jnp.float32)


def _matmul(h, w, br):
    n, d = h.shape
    return pl.pallas_call(
        _mm_body,
        grid=(n // br,),
        in_specs=[pl.BlockSpec((br, d), lambda i: (i, 0)),
                  pl.BlockSpec((d, d), lambda i: (0, 0))],
        out_specs=pl.BlockSpec((br, d), lambda i: (i, 0)),
        out_shape=jax.ShapeDtypeStruct((n, d), jnp.float32),
    )(h, w)


def _gru_body(p_ref, h_ref, wih_ref, whh_ref, bih_ref, bhh_ref, o_ref):
    d = h_ref.shape[1]
    agg = p_ref[0] + p_ref[1]
    h = h_ref[...]
    gi = jnp.dot(agg, wih_ref[...], preferred_element_type=jnp.float32)
    gi = gi + bih_ref[...]
    gh = jnp.dot(h, whh_ref[...], preferred_element_type=jnp.float32)
    gh = gh + bhh_ref[...]
    r = jax.nn.sigmoid(gi[:, :d] + gh[:, :d])
    z = jax.nn.sigmoid(gi[:, d:2 * d] + gh[:, d:2 * d])
    n = jnp.tanh(gi[:, 2 * d:] + r * gh[:, 2 * d:])
    o_ref[...] = (1.0 - z) * n + z * h


def _gru(partial, h, wih_t, whh_t, bih, bhh, br):
    n, d = h.shape
    return pl.pallas_call(
        _gru_body,
        grid=(n // br,),
        in_specs=[pl.BlockSpec((2, br, d), lambda i: (0, i, 0)),
                  pl.BlockSpec((br, d), lambda i: (i, 0)),
                  pl.BlockSpec((d, 3 * d), lambda i: (0, 0)),
                  pl.BlockSpec((d, 3 * d), lambda i: (0, 0)),
                  pl.BlockSpec((1, 3 * d), lambda i: (0, 0)),
                  pl.BlockSpec((1, 3 * d), lambda i: (0, 0))],
        out_specs=pl.BlockSpec((br, d), lambda i: (i, 0)),
        out_shape=jax.ShapeDtypeStruct((n, d), jnp.float32),
    )(partial, h, wih_t, whh_t, bih, bhh)


ZROWS = 16


def _seg_sum_body(n_nodes, d, base, rem,
                  m_hbm, src_hbm, dst_hbm, out_hbm,
                  src_v, dst_v, rows_a, rows_b, zbuf, acc, sem_a, sem_b):
    c = lax.axis_index("c")
    s = lax.axis_index("s")
    wid = c * NS + s

    zeros16 = jnp.zeros((16,), jnp.float32)

    def zrow(i, carry):
        def zcol(j, carry2):
            zbuf[i, pl.ds(j * 16, 16)] = zeros16
            return carry2
        return lax.fori_loop(0, d // 16, zcol, carry)

    lax.fori_loop(0, ZROWS, zrow, 0)
    for k in range(base // ZROWS):
        pltpu.sync_copy(zbuf, acc.at[pl.ds(s * base + k * ZROWS, ZROWS)])

    @pl.when(s == NS - 1)
    def _():
        pltpu.sync_copy(zbuf.at[pl.ds(0, rem)],
                        acc.at[pl.ds(NS * base, rem)])

    plsc.subcore_barrier()

    for t in range(2):
        pltpu.sync_copy(src_hbm.at[wid, t], src_v)
        pltpu.sync_copy(dst_hbm.at[wid, t], dst_v)
        def chunk_body(j, carry):
            pltpu.async_copy(m_hbm.at[src_v.at[j]], rows_a, sem_a).wait()
            pltpu.sync_copy(rows_a, acc.at[dst_v.at[j]], add=True)
            return carry

        lax.fori_loop(0, HALF, chunk_body, 0)
    plsc.subcore_barrier()

    pltpu.sync_copy(acc.at[pl.ds(s * base, base)],
                    out_hbm.at[c, pl.ds(s * base, base)])

    @pl.when(s == NS - 1)
    def _():
        pltpu.sync_copy(acc.at[pl.ds(NS * base, rem)],
                        out_hbm.at[c, pl.ds(NS * base, rem)])


def _seg_sum(m, src_r, dst_r):
    n_nodes, d = m.shape
    base = (n_nodes // (NS * 8)) * 8
    rem = n_nodes - NS * base
    assert base % ZROWS == 0 and rem <= ZROWS
    acc_rows = n_nodes + 8
    mesh = plsc.VectorSubcoreMesh(core_axis_name="c", subcore_axis_name="s")
    f = pl.kernel(
        functools.partial(_seg_sum_body, n_nodes, d, base, rem),
        out_type=jax.ShapeDtypeStruct((NC, n_nodes, d), jnp.float32),
        mesh=mesh,
        scratch_types=[
            pltpu.VMEM((HALF, CHUNK), jnp.int32),
            pltpu.VMEM((HALF, CHUNK), jnp.int32),
            pltpu.VMEM((CHUNK, d), jnp.float32),
            pltpu.VMEM((CHUNK, d), jnp.float32),
            pltpu.VMEM((ZROWS, d), jnp.float32),
            pltpu.VMEM_SHARED((acc_rows, d), jnp.float32),
            pltpu.SemaphoreType.DMA,
            pltpu.SemaphoreType.DMA,
        ],
    )
    return f(m, src_r, dst_r)


def _gather_body(h_hbm, map_hbm, out_hbm, idx_v, rows_v, sem):
    c = lax.axis_index("c")
    s = lax.axis_index("s")
    wid = c * NS + s
    pltpu.sync_copy(map_hbm.at[wid], idx_v)
    pltpu.async_copy(h_hbm.at[idx_v], rows_v, sem).wait()
    pltpu.sync_copy(rows_v, out_hbm.at[wid])


def _gather_clicked(h, mapping_idx):
    batch, num_clicked = mapping_idx.shape
    d = h.shape[1]
    mesh = plsc.VectorSubcoreMesh(core_axis_name="c", subcore_axis_name="s")
    f = pl.kernel(
        _gather_body,
        out_type=jax.ShapeDtypeStruct((batch, num_clicked, d), jnp.float32),
        mesh=mesh,
        scratch_types=[
            pltpu.VMEM((num_clicked,), jnp.int32),
            pltpu.VMEM((num_clicked, d), jnp.float32),
            pltpu.SemaphoreType.DMA,
        ],
    )
    return f(h, mapping_idx)


@jax.jit
def kernel(x, weight, w_ih, w_hh, b_ih, b_hh, edge_index, mapping_idx):
    n_nodes, d = x.shape
    n_edges = edge_index.shape[1]
    ept = 2 * HALF * CHUNK
    n_pad = NW * ept - n_edges

    pad_iota = jnp.arange(n_pad, dtype=jnp.int32)
    src_r = jnp.concatenate(
        [edge_index[0], pad_iota % jnp.int32(n_nodes)]
    ).reshape(NW, 2, HALF, CHUNK)
    dst_r = jnp.concatenate(
        [edge_index[1], PAD_DST + (pad_iota % jnp.int32(8))]
    ).reshape(NW, 2, HALF, CHUNK)
    wih_t = w_ih.T
    whh_t = w_hh.T
    bih = b_ih.reshape(1, 3 * d)
    bhh = b_hh.reshape(1, 3 * d)

    br = 2000
    h = x
    for i in range(N_LAYERS):
        m = _matmul(h, weight[i], br)
        partial = _seg_sum(m, src_r, dst_r)
        h = _gru(partial, h, wih_t, whh_t, bih, bhh, br)
    return _gather_clicked(h, mapping_idx)


# speedup vs baseline: 3.9073x; 1.4671x over previous
import functools

import jax
import jax.numpy as jnp
from jax import lax
from jax.experimental import pallas as pl
from jax.experimental.pallas import tpu as pltpu
from jax.experimental.pallas import tpu_sc as plsc

N_LAYERS = 3
NC, NS = 2, 16
NW = NC * NS
CHUNK = 128
HALF = 40
PAD_DST = 10000


def _mm_body(h_ref, w_ref, o_ref):
    o_ref[...] = jnp.dot(h_ref[...], w_ref[...],
                         preferred_element_type=jnp.float32)


def _matmul(h, w, br):
    n, d = h.shape
    return pl.pallas_call(
        _mm_body,
        grid=(n // br,),
        in_specs=[pl.BlockSpec((br, d), lambda i: (i, 0)),
                  pl.BlockSpec((d, d), lambda i: (0, 0))],
        out_specs=pl.BlockSpec((br, d), lambda i: (i, 0)),
        out_shape=jax.ShapeDtypeStruct((n, d), jnp.float32),
    )(h, w)


def _gru_body(p_ref, h_ref, wih_ref, whh_ref, bih_ref, bhh_ref, o_ref):
    d = h_ref.shape[1]
    agg = p_ref[0] + p_ref[1]
    h = h_ref[...]
    gi = jnp.dot(agg, wih_ref[...], preferred_element_type=jnp.float32)
    gi = gi + bih_ref[...]
    gh = jnp.dot(h, whh_ref[...], preferred_element_type=jnp.float32)
    gh = gh + bhh_ref[...]
    r = jax.nn.sigmoid(gi[:, :d] + gh[:, :d])
    z = jax.nn.sigmoid(gi[:, d:2 * d] + gh[:, d:2 * d])
    n = jnp.tanh(gi[:, 2 * d:] + r * gh[:, 2 * d:])
    o_ref[...] = (1.0 - z) * n + z * h


def _gru(partial, h, wih_t, whh_t, bih, bhh, br):
    n, d = h.shape
    return pl.pallas_call(
        _gru_body,
        grid=(n // br,),
        in_specs=[pl.BlockSpec((2, br, d), lambda i: (0, i, 0)),
                  pl.BlockSpec((br, d), lambda i: (i, 0)),
                  pl.BlockSpec((d, 3 * d), lambda i: (0, 0)),
                  pl.BlockSpec((d, 3 * d), lambda i: (0, 0)),
                  pl.BlockSpec((1, 3 * d), lambda i: (0, 0)),
                  pl.BlockSpec((1, 3 * d), lambda i: (0, 0))],
        out_specs=pl.BlockSpec((br, d), lambda i: (i, 0)),
        out_shape=jax.ShapeDtypeStruct((n, d), jnp.float32),
    )(partial, h, wih_t, whh_t, bih, bhh)


ZROWS = 16


def _seg_sum_body(n_nodes, d, base, rem,
                  m_hbm, src_hbm, dst_hbm, out_hbm,
                  src_v, dst_v, rows_a, rows_b, zbuf, acc, sem_a, sem_b):
    c = lax.axis_index("c")
    s = lax.axis_index("s")
    wid = c * NS + s

    zeros16 = jnp.zeros((16,), jnp.float32)

    def zrow(i, carry):
        def zcol(j, carry2):
            zbuf[i, pl.ds(j * 16, 16)] = zeros16
            return carry2
        return lax.fori_loop(0, d // 16, zcol, carry)

    lax.fori_loop(0, ZROWS, zrow, 0)
    for k in range(base // ZROWS):
        pltpu.sync_copy(zbuf, acc.at[pl.ds(s * base + k * ZROWS, ZROWS)])

    @pl.when(s == NS - 1)
    def _():
        pltpu.sync_copy(zbuf.at[pl.ds(0, rem)],
                        acc.at[pl.ds(NS * base, rem)])

    plsc.subcore_barrier()

    for t in range(2):
        pltpu.sync_copy(src_hbm.at[wid, t], src_v)
        pltpu.sync_copy(dst_hbm.at[wid, t], dst_v)
        pltpu.async_copy(m_hbm.at[src_v.at[0]], rows_a, sem_a)
        pltpu.async_copy(m_hbm.at[src_v.at[1]], rows_b, sem_b)

        def pair_body(i, carry):
            j = 2 * i
            for off, buf, sem in ((0, rows_a, sem_a), (1, rows_b, sem_b)):
                pltpu.make_async_copy(m_hbm.at[src_v.at[j + off]], buf,
                                      sem).wait()
                pltpu.sync_copy(buf, acc.at[dst_v.at[j + off]], add=True)

                @pl.when(j + off + 2 < HALF)
                def _():
                    pltpu.async_copy(m_hbm.at[src_v.at[j + off + 2]], buf, sem)
            return carry

        lax.fori_loop(0, HALF // 2, pair_body, 0)
    plsc.subcore_barrier()

    pltpu.sync_copy(acc.at[pl.ds(s * base, base)],
                    out_hbm.at[c, pl.ds(s * base, base)])

    @pl.when(s == NS - 1)
    def _():
        pltpu.sync_copy(acc.at[pl.ds(NS * base, rem)],
                        out_hbm.at[c, pl.ds(NS * base, rem)])


def _seg_sum(m, src_r, dst_r):
    n_nodes, d = m.shape
    base = (n_nodes // (NS * 8)) * 8
    rem = n_nodes - NS * base
    assert base % ZROWS == 0 and rem <= ZROWS
    acc_rows = n_nodes + 8
    mesh = plsc.VectorSubcoreMesh(core_axis_name="c", subcore_axis_name="s")
    f = pl.kernel(
        functools.partial(_seg_sum_body, n_nodes, d, base, rem),
        out_type=jax.ShapeDtypeStruct((NC, n_nodes, d), jnp.float32),
        mesh=mesh,
        scratch_types=[
            pltpu.VMEM((HALF, CHUNK), jnp.int32),
            pltpu.VMEM((HALF, CHUNK), jnp.int32),
            pltpu.VMEM((CHUNK, d), jnp.float32),
            pltpu.VMEM((CHUNK, d), jnp.float32),
            pltpu.VMEM((ZROWS, d), jnp.float32),
            pltpu.VMEM_SHARED((acc_rows, d), jnp.float32),
            pltpu.SemaphoreType.DMA,
            pltpu.SemaphoreType.DMA,
        ],
    )
    return f(m, src_r, dst_r)


def _gather_body(h_hbm, map_hbm, out_hbm, idx_v, rows_v, sem):
    c = lax.axis_index("c")
    s = lax.axis_index("s")
    wid = c * NS + s
    pltpu.sync_copy(map_hbm.at[wid], idx_v)
    pltpu.async_copy(h_hbm.at[idx_v], rows_v, sem).wait()
    pltpu.sync_copy(rows_v, out_hbm.at[wid])


def _gather_clicked(h, mapping_idx):
    batch, num_clicked = mapping_idx.shape
    d = h.shape[1]
    mesh = plsc.VectorSubcoreMesh(core_axis_name="c", subcore_axis_name="s")
    f = pl.kernel(
        _gather_body,
        out_type=jax.ShapeDtypeStruct((batch, num_clicked, d), jnp.float32),
        mesh=mesh,
        scratch_types=[
            pltpu.VMEM((num_clicked,), jnp.int32),
            pltpu.VMEM((num_clicked, d), jnp.float32),
            pltpu.SemaphoreType.DMA,
        ],
    )
    return f(h, mapping_idx)


@jax.jit
def kernel(x, weight, w_ih, w_hh, b_ih, b_hh, edge_index, mapping_idx):
    n_nodes, d = x.shape
    n_edges = edge_index.shape[1]
    ept = 2 * HALF * CHUNK
    n_pad = NW * ept - n_edges

    pad_iota = jnp.arange(n_pad, dtype=jnp.int32)
    src_r = jnp.concatenate(
        [edge_index[0], pad_iota % jnp.int32(n_nodes)]
    ).reshape(NW, 2, HALF, CHUNK)
    dst_r = jnp.concatenate(
        [edge_index[1], PAD_DST + (pad_iota % jnp.int32(8))]
    ).reshape(NW, 2, HALF, CHUNK)
    wih_t = w_ih.T
    whh_t = w_hh.T
    bih = b_ih.reshape(1, 3 * d)
    bhh = b_hh.reshape(1, 3 * d)

    br = 2000
    h = x
    for i in range(N_LAYERS):
        m = _matmul(h, weight[i], br)
        partial = _seg_sum(m, src_r, dst_r)
        h = _gru(partial, h, wih_t, whh_t, bih, bhh, br)
    return _gather_clicked(h, mapping_idx)


# speedup vs baseline: 4.1985x; 1.0745x over previous
import functools

import jax
import jax.numpy as jnp
from jax import lax
from jax.experimental import pallas as pl
from jax.experimental.pallas import tpu as pltpu
from jax.experimental.pallas import tpu_sc as plsc

N_LAYERS = 3
NC, NS = 2, 16
NW = NC * NS
CHUNK = 128
HALF = 40
PAD_DST = 10000


def _mm_body(h_ref, w_ref, o_ref):
    o_ref[...] = jnp.dot(h_ref[...], w_ref[...],
                         preferred_element_type=jnp.float32)


def _matmul(h, w, br):
    n, d = h.shape
    return pl.pallas_call(
        _mm_body,
        grid=(n // br,),
        in_specs=[pl.BlockSpec((br, d), lambda i: (i, 0)),
                  pl.BlockSpec((d, d), lambda i: (0, 0))],
        out_specs=pl.BlockSpec((br, d), lambda i: (i, 0)),
        out_shape=jax.ShapeDtypeStruct((n, d), jnp.float32),
    )(h, w)


def _gru_body(p_ref, h_ref, w_ref, wih_ref, whh_ref, bih_ref, bhh_ref, o_ref):
    d = h_ref.shape[1]
    agg = jnp.dot(p_ref[0] + p_ref[1], w_ref[...],
                  preferred_element_type=jnp.float32)
    h = h_ref[...]
    gi = jnp.dot(agg, wih_ref[...], preferred_element_type=jnp.float32)
    gi = gi + bih_ref[...]
    gh = jnp.dot(h, whh_ref[...], preferred_element_type=jnp.float32)
    gh = gh + bhh_ref[...]
    r = jax.nn.sigmoid(gi[:, :d] + gh[:, :d])
    z = jax.nn.sigmoid(gi[:, d:2 * d] + gh[:, d:2 * d])
    n = jnp.tanh(gi[:, 2 * d:] + r * gh[:, 2 * d:])
    o_ref[...] = (1.0 - z) * n + z * h


def _gru(partial, h, w, wih_t, whh_t, bih, bhh, br):
    n, d = h.shape
    return pl.pallas_call(
        _gru_body,
        grid=(n // br,),
        in_specs=[pl.BlockSpec((2, br, d), lambda i: (0, i, 0)),
                  pl.BlockSpec((br, d), lambda i: (i, 0)),
                  pl.BlockSpec((d, d), lambda i: (0, 0)),
                  pl.BlockSpec((d, 3 * d), lambda i: (0, 0)),
                  pl.BlockSpec((d, 3 * d), lambda i: (0, 0)),
                  pl.BlockSpec((1, 3 * d), lambda i: (0, 0)),
                  pl.BlockSpec((1, 3 * d), lambda i: (0, 0))],
        out_specs=pl.BlockSpec((br, d), lambda i: (i, 0)),
        out_shape=jax.ShapeDtypeStruct((n, d), jnp.float32),
    )(partial, h, w, wih_t, whh_t, bih, bhh)


ZROWS = 16


def _seg_sum_body(n_nodes, d, base, rem,
                  m_hbm, src_hbm, dst_hbm, out_hbm,
                  src_v, dst_v, rows_a, rows_b, zbuf, acc, sem_a, sem_b):
    c = lax.axis_index("c")
    s = lax.axis_index("s")
    wid = c * NS + s

    zeros16 = jnp.zeros((16,), jnp.float32)

    def zrow(i, carry):
        def zcol(j, carry2):
            zbuf[i, pl.ds(j * 16, 16)] = zeros16
            return carry2
        return lax.fori_loop(0, d // 16, zcol, carry)

    lax.fori_loop(0, ZROWS, zrow, 0)
    for k in range(base // ZROWS):
        pltpu.async_copy(zbuf, acc.at[pl.ds(s * base + k * ZROWS, ZROWS)],
                         sem_a)

    @pl.when(s == NS - 1)
    def _():
        pltpu.async_copy(zbuf.at[pl.ds(0, rem)],
                         acc.at[pl.ds(NS * base, rem)], sem_a)

    for k in range(base // ZROWS):
        pltpu.make_async_copy(zbuf, acc.at[pl.ds(s * base + k * ZROWS,
                                                 ZROWS)], sem_a).wait()

    @pl.when(s == NS - 1)
    def _():
        pltpu.make_async_copy(zbuf.at[pl.ds(0, rem)],
                              acc.at[pl.ds(NS * base, rem)], sem_a).wait()

    plsc.subcore_barrier()

    for t in range(2):
        pltpu.sync_copy(src_hbm.at[wid, t], src_v)
        pltpu.sync_copy(dst_hbm.at[wid, t], dst_v)
        pltpu.async_copy(m_hbm.at[src_v.at[0]], rows_a, sem_a)
        pltpu.async_copy(m_hbm.at[src_v.at[1]], rows_b, sem_b)

        def pair_body(i, carry):
            j = 2 * i
            for off, buf, sem in ((0, rows_a, sem_a), (1, rows_b, sem_b)):
                pltpu.make_async_copy(m_hbm.at[src_v.at[j + off]], buf,
                                      sem).wait()
                pltpu.sync_copy(buf, acc.at[dst_v.at[j + off]], add=True)

                @pl.when(j + off + 2 < HALF)
                def _():
                    pltpu.async_copy(m_hbm.at[src_v.at[j + off + 2]], buf, sem)
            return carry

        lax.fori_loop(0, HALF // 2, pair_body, 0)
    plsc.subcore_barrier()

    pltpu.sync_copy(acc.at[pl.ds(s * base, base)],
                    out_hbm.at[c, pl.ds(s * base, base)])

    @pl.when(s == NS - 1)
    def _():
        pltpu.sync_copy(acc.at[pl.ds(NS * base, rem)],
                        out_hbm.at[c, pl.ds(NS * base, rem)])


def _seg_sum(m, src_r, dst_r):
    n_nodes, d = m.shape
    base = (n_nodes // (NS * 8)) * 8
    rem = n_nodes - NS * base
    assert base % ZROWS == 0 and rem <= ZROWS
    acc_rows = n_nodes + 8
    mesh = plsc.VectorSubcoreMesh(core_axis_name="c", subcore_axis_name="s")
    f = pl.kernel(
        functools.partial(_seg_sum_body, n_nodes, d, base, rem),
        out_type=jax.ShapeDtypeStruct((NC, n_nodes, d), jnp.float32),
        mesh=mesh,
        scratch_types=[
            pltpu.VMEM((HALF, CHUNK), jnp.int32),
            pltpu.VMEM((HALF, CHUNK), jnp.int32),
            pltpu.VMEM((CHUNK, d), jnp.float32),
            pltpu.VMEM((CHUNK, d), jnp.float32),
            pltpu.VMEM((ZROWS, d), jnp.float32),
            pltpu.VMEM_SHARED((acc_rows, d), jnp.float32),
            pltpu.SemaphoreType.DMA,
            pltpu.SemaphoreType.DMA,
        ],
    )
    return f(m, src_r, dst_r)


def _gather_body(h_hbm, map_hbm, out_hbm, idx_v, rows_v, sem):
    c = lax.axis_index("c")
    s = lax.axis_index("s")
    wid = c * NS + s
    pltpu.sync_copy(map_hbm.at[wid], idx_v)
    pltpu.async_copy(h_hbm.at[idx_v], rows_v, sem).wait()
    pltpu.sync_copy(rows_v, out_hbm.at[wid])


def _gather_clicked(h, mapping_idx):
    batch, num_clicked = mapping_idx.shape
    d = h.shape[1]
    mesh = plsc.VectorSubcoreMesh(core_axis_name="c", subcore_axis_name="s")
    f = pl.kernel(
        _gather_body,
        out_type=jax.ShapeDtypeStruct((batch, num_clicked, d), jnp.float32),
        mesh=mesh,
        scratch_types=[
            pltpu.VMEM((num_clicked,), jnp.int32),
            pltpu.VMEM((num_clicked, d), jnp.float32),
            pltpu.SemaphoreType.DMA,
        ],
    )
    return f(h, mapping_idx)


@jax.jit
def kernel(x, weight, w_ih, w_hh, b_ih, b_hh, edge_index, mapping_idx):
    n_nodes, d = x.shape
    n_edges = edge_index.shape[1]
    ept = 2 * HALF * CHUNK
    n_pad = NW * ept - n_edges

    pad_iota = jnp.arange(n_pad, dtype=jnp.int32)
    src_r = jnp.concatenate(
        [edge_index[0], pad_iota % jnp.int32(n_nodes)]
    ).reshape(NW, 2, HALF, CHUNK)
    dst_r = jnp.concatenate(
        [edge_index[1], PAD_DST + (pad_iota % jnp.int32(8))]
    ).reshape(NW, 2, HALF, CHUNK)
    wih_t = w_ih.T
    whh_t = w_hh.T
    bih = b_ih.reshape(1, 3 * d)
    bhh = b_hh.reshape(1, 3 * d)

    br = 2000
    h = x
    for i in range(N_LAYERS):
        partial = _seg_sum(h, src_r, dst_r)
        h = _gru(partial, h, weight[i], wih_t, whh_t, bih, bhh, br)
    return _gather_clicked(h, mapping_idx)
